# baseline (speedup 1.0000x reference)
import jax
import jax.numpy as jnp
from jax.experimental import pallas as pl
from jax.experimental.pallas import tpu as pltpu

TOTAL = 32768
BATCH = 16
VAL = 2 * TOTAL


def _body(a_ref, aoff_ref, b_ref, boff_ref, id_ref,
          vals_ref, lens_ref, offs_ref,
          sem_a, sem_b, sem_id):
    cp_a = pltpu.make_async_copy(a_ref, vals_ref.at[pl.ds(0, TOTAL)], sem_a)
    cp_b = pltpu.make_async_copy(b_ref, vals_ref.at[pl.ds(TOTAL, TOTAL)], sem_b)
    cp_i = pltpu.make_async_copy(id_ref, vals_ref.at[pl.ds(VAL, BATCH)], sem_id)
    cp_a.start()
    cp_b.start()
    cp_i.start()

    aoff = aoff_ref[...]
    boff = boff_ref[...]
    a_lo = aoff[0:BATCH]
    a_hi = aoff[1:BATCH + 1]
    b_lo = boff[0:BATCH]
    b_hi = boff[1:BATCH + 1]
    ramp = jax.lax.broadcasted_iota(jnp.int32, (BATCH,), 0)
    lens_ref[...] = jnp.concatenate(
        [a_hi - a_lo, b_hi - b_lo, jnp.ones((BATCH,), jnp.int32)])
    offs_ref[...] = jnp.concatenate(
        [aoff, b_hi + TOTAL, ramp + (VAL + 1)])

    cp_a.wait()
    cp_b.wait()
    cp_i.wait()


def kernel(feat_a__values, feat_a__offsets, feat_b__values, feat_b__offsets, id):
    out = pl.pallas_call(
        _body,
        out_shape=(
            jax.ShapeDtypeStruct((VAL + BATCH,), jnp.float32),
            jax.ShapeDtypeStruct((3 * BATCH,), jnp.int32),
            jax.ShapeDtypeStruct((3 * BATCH + 1,), jnp.int32),
        ),
        out_specs=(
            pl.BlockSpec(memory_space=pl.ANY),
            pl.BlockSpec(memory_space=pltpu.VMEM),
            pl.BlockSpec(memory_space=pltpu.VMEM),
        ),
        scratch_shapes=[
            pltpu.SemaphoreType.DMA,
            pltpu.SemaphoreType.DMA,
            pltpu.SemaphoreType.DMA,
        ],
    )(feat_a__values, feat_a__offsets, feat_b__values, feat_b__offsets, id)
    return tuple(out)
